# Initial kernel scaffold; baseline (speedup 1.0000x reference)
import jax
import jax.numpy as jnp
from jax.experimental import pallas as pl


def kernel(x, pe):
    raise NotImplementedError("write your pallas kernel here")


# speedup vs baseline: 1.9103x; 1.9103x over previous
import jax
import jax.numpy as jnp
from jax.experimental import pallas as pl

_BLK = 512


def _pe_add_kernel(x_ref, pe_ref, o_ref):
    o_ref[...] = x_ref[...] + pe_ref[...][None, :, :]


def kernel(x, pe):
    B, L, E = x.shape
    pe_rows = jax.lax.slice(pe, (1, 0), (1 + L, E))
    return pl.pallas_call(
        _pe_add_kernel,
        grid=(B, L // _BLK),
        in_specs=[
            pl.BlockSpec((1, _BLK, E), lambda i, j: (i, j, 0)),
            pl.BlockSpec((_BLK, E), lambda i, j: (j, 0)),
        ],
        out_specs=pl.BlockSpec((1, _BLK, E), lambda i, j: (i, j, 0)),
        out_shape=jax.ShapeDtypeStruct((B, L, E), x.dtype),
    )(x, pe_rows)


# speedup vs baseline: 2.3698x; 1.2405x over previous
import jax
import jax.numpy as jnp
from jax.experimental import pallas as pl

_BLK = 256


def _pe_add_kernel(x_ref, pe_ref, o_ref):
    o_ref[...] = x_ref[...] + pe_ref[...][None, :, :]


def kernel(x, pe):
    B, L, E = x.shape
    pe_rows = jax.lax.slice(pe, (1, 0), (1 + L, E))
    return pl.pallas_call(
        _pe_add_kernel,
        grid=(L // _BLK,),
        in_specs=[
            pl.BlockSpec((B, _BLK, E), lambda j: (0, j, 0)),
            pl.BlockSpec((_BLK, E), lambda j: (j, 0)),
        ],
        out_specs=pl.BlockSpec((B, _BLK, E), lambda j: (0, j, 0)),
        out_shape=jax.ShapeDtypeStruct((B, L, E), x.dtype),
    )(x, pe_rows)


# speedup vs baseline: 2.3713x; 1.0006x over previous
import jax
import jax.numpy as jnp
from jax.experimental import pallas as pl
from jax.experimental.pallas import tpu as pltpu

_BLK = 256


def _pe_add_kernel(x_ref, pe_ref, o_ref):
    o_ref[...] = x_ref[...] + pe_ref[...][None, :, :]


def kernel(x, pe):
    B, L, E = x.shape
    pe_rows = jax.lax.slice(pe, (1, 0), (1 + L, E))
    return pl.pallas_call(
        _pe_add_kernel,
        grid=(L // _BLK,),
        in_specs=[
            pl.BlockSpec((B, _BLK, E), lambda j: (0, j, 0)),
            pl.BlockSpec((_BLK, E), lambda j: (j, 0)),
        ],
        out_specs=pl.BlockSpec((B, _BLK, E), lambda j: (0, j, 0)),
        out_shape=jax.ShapeDtypeStruct((B, L, E), x.dtype),
        compiler_params=pltpu.CompilerParams(
            dimension_semantics=("parallel",),
        ),
    )(x, pe_rows)
